# baseline (speedup 1.0000x reference)
import functools

import jax
import jax.numpy as jnp
from jax import lax
from jax.experimental import pallas as pl
from jax.experimental.pallas import tpu as pltpu
from jax.experimental.pallas import tpu_sc as plsc

N = 10000
E = 320000
D = 128
R = 16
NB = 8

NC = 2
NS = 16
NW = NC * NS
EPW = E // NW
C = 80
NCHUNK = EPW // C
NP = 10240
RPT = NP // NS
SST = 2000


def _xw_body(a_ref, v_ref, f_ref, out_ref):
    w = jax.lax.dot_general(a_ref[...], v_ref[...],
                            (((1,), (0,)), ((), ())),
                            preferred_element_type=jnp.float32)
    w = w.astype(jnp.bfloat16)
    f = f_ref[...].astype(jnp.bfloat16)
    for rr in range(R):
        out_ref[:, rr, :] = jnp.dot(f, w[rr],
                                    preferred_element_type=jnp.float32)


def _final_body(f_ref, w_ref, p_ref, out_ref):
    acc = p_ref[0] + p_ref[1] + jnp.dot(f_ref[...], w_ref[...],
                                        preferred_element_type=jnp.float32)
    out_ref[...] = jnp.maximum(acc, 0.0)


def _sc_body(src_hbm, rel_hbm, dst_hbm, xw_hbm, out_hbm,
             g_v, srcst_v, dst2_v, rows_a, rows_b, agg_sh,
             sem_ga, sem_gb, sem_sa, sem_sb):
    c = lax.axis_index("c")
    s = lax.axis_index("s")
    wid = c * NS + s

    zero16 = jnp.zeros((16,), jnp.float32)

    def zrow(i, carry):
        for j in range(D // 16):
            rows_a[i, pl.ds(j * 16, 16)] = zero16
        return carry

    lax.fori_loop(0, C, zrow, 0)
    for k in range(RPT // C):
        pltpu.sync_copy(rows_a, agg_sh.at[pl.ds(s * RPT + k * C, C)])

    pltpu.sync_copy(rel_hbm.at[pl.ds(wid * EPW, EPW)], g_v)
    pltpu.sync_copy(dst_hbm.at[wid], dst2_v)
    for h in range(EPW // SST):
        pltpu.sync_copy(src_hbm.at[pl.ds(wid * EPW + h * SST, SST)], srcst_v)

        def gstep(i, carry):
            sl = pl.ds(h * SST + i * 16, 16)
            g_v[sl] = srcst_v[pl.ds(i * 16, 16)] * R + g_v[sl]
            return carry

        lax.fori_loop(0, SST // 16, gstep, 0)
    plsc.subcore_barrier()

    def start_g(cidx, rows, sem):
        pltpu.async_copy(xw_hbm.at[g_v.at[pl.ds(cidx * C, C)]], rows, sem)

    def wait_g(cidx, rows, sem):
        pltpu.make_async_copy(xw_hbm.at[g_v.at[pl.ds(cidx * C, C)]], rows,
                              sem).wait()

    def start_s(cidx, rows, sem):
        pltpu.async_copy(rows, agg_sh.at[dst2_v.at[cidx]], sem, add=True)

    def wait_s(cidx, rows, sem):
        pltpu.make_async_copy(rows, agg_sh.at[dst2_v.at[cidx]], sem).wait()

    start_g(0, rows_a, sem_ga)

    def body(jj, carry):
        ca = 2 * jj
        cb = 2 * jj + 1
        start_g(cb, rows_b, sem_gb)
        wait_g(ca, rows_a, sem_ga)
        start_s(ca, rows_a, sem_sa)
        wait_g(cb, rows_b, sem_gb)
        start_s(cb, rows_b, sem_sb)
        wait_s(ca, rows_a, sem_sa)
        start_g(ca + 2, rows_a, sem_ga)
        wait_s(cb, rows_b, sem_sb)
        return carry

    lax.fori_loop(0, (NCHUNK - 1) // 2, body, 0)
    wait_g(NCHUNK - 1, rows_a, sem_ga)
    start_s(NCHUNK - 1, rows_a, sem_sa)
    wait_s(NCHUNK - 1, rows_a, sem_sa)
    plsc.subcore_barrier()

    for k in range(RPT // C):
        base = s * RPT + k * C
        pltpu.sync_copy(agg_sh.at[pl.ds(base, C)], rows_a)
        pltpu.sync_copy(rows_a, out_hbm.at[c, pl.ds(base, C)])


@functools.lru_cache(maxsize=None)
def _make_sc_call():
    return pl.kernel(
        _sc_body,
        mesh=plsc.VectorSubcoreMesh(core_axis_name="c", subcore_axis_name="s"),
        out_type=jax.ShapeDtypeStruct((NC, NP, D), jnp.float32),
        scratch_types=[
            pltpu.VMEM((EPW,), jnp.int32),
            pltpu.VMEM((SST,), jnp.int32),
            pltpu.VMEM((NCHUNK, C), jnp.int32),
            pltpu.VMEM((C, D), jnp.float32),
            pltpu.VMEM((C, D), jnp.float32),
            pltpu.VMEM_SHARED((NP, D), jnp.float32),
            pltpu.SemaphoreType.DMA,
            pltpu.SemaphoreType.DMA,
            pltpu.SemaphoreType.DMA,
            pltpu.SemaphoreType.DMA,
        ],
    )


def kernel(adj, feats, r, V0, a0, Wsl0, V1, a1, Wsl1):
    src = adj[0]
    dst = adj[1]

    BN = 1000
    xw = pl.pallas_call(
        _xw_body,
        grid=(N // BN,),
        in_specs=[
            pl.BlockSpec((R, NB), lambda i: (0, 0)),
            pl.BlockSpec((NB, D, D), lambda i: (0, 0, 0)),
            pl.BlockSpec((BN, D), lambda i: (i, 0)),
        ],
        out_specs=pl.BlockSpec((BN, R, D), lambda i: (i, 0, 0)),
        out_shape=jax.ShapeDtypeStruct((N, R, D), jnp.float32),
    )(a1, V1, feats)

    partials = _make_sc_call()(src, r, dst.reshape(NW, NCHUNK, C),
                               xw.reshape(N * R, D))

    out = pl.pallas_call(
        _final_body,
        grid=(N // BN,),
        in_specs=[
            pl.BlockSpec((BN, D), lambda i: (i, 0)),
            pl.BlockSpec((D, D), lambda i: (0, 0)),
            pl.BlockSpec((NC, BN, D), lambda i: (0, i, 0)),
        ],
        out_specs=pl.BlockSpec((BN, D), lambda i: (i, 0)),
        out_shape=jax.ShapeDtypeStruct((N, D), jnp.float32),
    )(feats, Wsl1, partials)
    return out


# speedup vs baseline: 1.1569x; 1.1569x over previous
import functools

import jax
import jax.numpy as jnp
from jax import lax
from jax.experimental import pallas as pl
from jax.experimental.pallas import tpu as pltpu
from jax.experimental.pallas import tpu_sc as plsc

N = 10000
E = 320000
D = 128
R = 16
NB = 8

NC = 2
NS = 16
NW = NC * NS
EPW = E // NW
C = 80
NCHUNK = EPW // C
NP = 10240
RPT = NP // NS
SST = 2000


def _xw_body(a_ref, v_ref, f_ref, out_ref):
    w = jax.lax.dot_general(a_ref[...], v_ref[...],
                            (((1,), (0,)), ((), ())),
                            preferred_element_type=jnp.float32)
    w = w.astype(jnp.bfloat16)
    f = f_ref[...].astype(jnp.bfloat16)
    for rr in range(R):
        out_ref[:, rr, :] = jnp.dot(f, w[rr],
                                    preferred_element_type=jnp.float32)


def _final_body(f_ref, w_ref, p_ref, out_ref):
    acc = p_ref[0] + p_ref[1] + jnp.dot(f_ref[...], w_ref[...],
                                        preferred_element_type=jnp.float32)
    out_ref[...] = jnp.maximum(acc, 0.0)


def _sc_body(src_hbm, rel_hbm, dst_hbm, xw_hbm, out_hbm,
             g_v, srcst_v, dst2_v, rows_a, rows_b, agg_sh,
             sem_ga, sem_gb, sem_sa, sem_sb):
    c = lax.axis_index("c")
    s = lax.axis_index("s")
    wid = c * NS + s

    zero16 = jnp.zeros((16,), jnp.float32)

    def zrow(i, carry):
        for j in range(D // 16):
            rows_a[i, pl.ds(j * 16, 16)] = zero16
        return carry

    lax.fori_loop(0, C, zrow, 0)
    for k in range(RPT // C):
        pltpu.sync_copy(rows_a, agg_sh.at[pl.ds(s * RPT + k * C, C)])

    pltpu.sync_copy(rel_hbm.at[pl.ds(wid * EPW, EPW)], g_v)
    pltpu.sync_copy(dst_hbm.at[wid], dst2_v)
    for h in range(EPW // SST):
        pltpu.sync_copy(src_hbm.at[pl.ds(wid * EPW + h * SST, SST)], srcst_v)

        def gstep(i, carry):
            sl = pl.ds(h * SST + i * 16, 16)
            g_v[sl] = srcst_v[pl.ds(i * 16, 16)] * R + g_v[sl]
            return carry

        lax.fori_loop(0, SST // 16, gstep, 0)
    plsc.subcore_barrier()

    def start_g(cidx, rows, sem):
        pltpu.async_copy(xw_hbm.at[g_v.at[pl.ds(cidx * C, C)]], rows, sem)

    def wait_g(cidx, rows, sem):
        pltpu.make_async_copy(xw_hbm.at[g_v.at[pl.ds(cidx * C, C)]], rows,
                              sem).wait()

    def start_s(cidx, rows, sem):
        pltpu.async_copy(rows, agg_sh.at[dst2_v.at[cidx]], sem, add=True)

    def wait_s(cidx, rows, sem):
        pltpu.make_async_copy(rows, agg_sh.at[dst2_v.at[cidx]], sem).wait()

    start_g(0, rows_a, sem_ga)

    def scatter(cidx, rows):
        pltpu.sync_copy(rows, agg_sh.at[dst2_v.at[cidx]], add=True)

    def body(jj, carry):
        ca = 2 * jj
        cb = 2 * jj + 1
        start_g(cb, rows_b, sem_gb)
        wait_g(ca, rows_a, sem_ga)
        scatter(ca, rows_a)
        start_g(ca + 2, rows_a, sem_ga)
        wait_g(cb, rows_b, sem_gb)
        scatter(cb, rows_b)
        return carry

    lax.fori_loop(0, (NCHUNK - 1) // 2, body, 0)
    wait_g(NCHUNK - 1, rows_a, sem_ga)
    scatter(NCHUNK - 1, rows_a)
    plsc.subcore_barrier()

    for k in range(RPT // C):
        base = s * RPT + k * C
        pltpu.sync_copy(agg_sh.at[pl.ds(base, C)], rows_a)
        pltpu.sync_copy(rows_a, out_hbm.at[c, pl.ds(base, C)])


@functools.lru_cache(maxsize=None)
def _make_sc_call():
    return pl.kernel(
        _sc_body,
        mesh=plsc.VectorSubcoreMesh(core_axis_name="c", subcore_axis_name="s"),
        out_type=jax.ShapeDtypeStruct((NC, NP, D), jnp.float32),
        scratch_types=[
            pltpu.VMEM((EPW,), jnp.int32),
            pltpu.VMEM((SST,), jnp.int32),
            pltpu.VMEM((NCHUNK, C), jnp.int32),
            pltpu.VMEM((C, D), jnp.float32),
            pltpu.VMEM((C, D), jnp.float32),
            pltpu.VMEM_SHARED((NP, D), jnp.float32),
            pltpu.SemaphoreType.DMA,
            pltpu.SemaphoreType.DMA,
            pltpu.SemaphoreType.DMA,
            pltpu.SemaphoreType.DMA,
        ],
    )


def kernel(adj, feats, r, V0, a0, Wsl0, V1, a1, Wsl1):
    src = adj[0]
    dst = adj[1]

    BN = 1000
    xw = pl.pallas_call(
        _xw_body,
        grid=(N // BN,),
        in_specs=[
            pl.BlockSpec((R, NB), lambda i: (0, 0)),
            pl.BlockSpec((NB, D, D), lambda i: (0, 0, 0)),
            pl.BlockSpec((BN, D), lambda i: (i, 0)),
        ],
        out_specs=pl.BlockSpec((BN, R, D), lambda i: (i, 0, 0)),
        out_shape=jax.ShapeDtypeStruct((N, R, D), jnp.float32),
    )(a1, V1, feats)

    partials = _make_sc_call()(src, r, dst.reshape(NW, NCHUNK, C),
                               xw.reshape(N * R, D))

    out = pl.pallas_call(
        _final_body,
        grid=(N // BN,),
        in_specs=[
            pl.BlockSpec((BN, D), lambda i: (i, 0)),
            pl.BlockSpec((D, D), lambda i: (0, 0)),
            pl.BlockSpec((NC, BN, D), lambda i: (0, i, 0)),
        ],
        out_specs=pl.BlockSpec((BN, D), lambda i: (i, 0)),
        out_shape=jax.ShapeDtypeStruct((N, D), jnp.float32),
    )(feats, Wsl1, partials)
    return out


# speedup vs baseline: 1.3214x; 1.1421x over previous
import functools

import jax
import jax.numpy as jnp
from jax import lax
from jax.experimental import pallas as pl
from jax.experimental.pallas import tpu as pltpu
from jax.experimental.pallas import tpu_sc as plsc

N = 10000
E = 320000
D = 128
R = 16
NB = 8

NC = 2
NS = 16
NW = NC * NS
EPW = E // NW
C = 80
NCHUNK = EPW // C
NP = 10240
RPT = NP // NS
SST = 2000


def _xw_body(a_ref, v_ref, f_ref, out_ref):
    w = jax.lax.dot_general(a_ref[...], v_ref[...],
                            (((1,), (0,)), ((), ())),
                            preferred_element_type=jnp.float32)
    w = w.astype(jnp.bfloat16)
    f = f_ref[...].astype(jnp.bfloat16)
    for rr in range(R):
        out_ref[rr] = jnp.dot(f, w[rr], preferred_element_type=jnp.float32)


def _final_body(f_ref, w_ref, p_ref, out_ref):
    acc = p_ref[0] + p_ref[1] + jnp.dot(f_ref[...], w_ref[...],
                                        preferred_element_type=jnp.float32)
    out_ref[...] = jnp.maximum(acc, 0.0)


def _sc_body(src_hbm, dst_hbm, rel_hbm, xw_hbm, out_hbm,
             g_v, srcst_v, dst_v, rows_a, rows_b, agg_sh,
             sem_ga, sem_gb, sem_sa, sem_sb):
    c = lax.axis_index("c")
    s = lax.axis_index("s")
    wid = c * NS + s

    zero16 = jnp.zeros((16,), jnp.float32)

    def zrow(i, carry):
        for j in range(D // 16):
            rows_a[i, pl.ds(j * 16, 16)] = zero16
        return carry

    lax.fori_loop(0, C, zrow, 0)
    for k in range(RPT // C):
        pltpu.sync_copy(rows_a, agg_sh.at[pl.ds(s * RPT + k * C, C)])

    pltpu.sync_copy(rel_hbm.at[pl.ds(wid * EPW, EPW)], g_v)
    pltpu.sync_copy(dst_hbm.at[pl.ds(wid * EPW, EPW)], dst_v)
    for h in range(EPW // SST):
        pltpu.sync_copy(src_hbm.at[pl.ds(wid * EPW + h * SST, SST)], srcst_v)

        def gstep(i, carry):
            sl = pl.ds(h * SST + i * 16, 16)
            g_v[sl] = g_v[sl] * N + srcst_v[pl.ds(i * 16, 16)]
            return carry

        lax.fori_loop(0, SST // 16, gstep, 0)
    plsc.subcore_barrier()

    def start_g(cidx, rows, sem):
        pltpu.async_copy(xw_hbm.at[g_v.at[pl.ds(cidx * C, C)]], rows, sem)

    def wait_g(cidx, rows, sem):
        pltpu.make_async_copy(xw_hbm.at[g_v.at[pl.ds(cidx * C, C)]], rows,
                              sem).wait()

    def start_s(cidx, rows, sem):
        pltpu.async_copy(rows, agg_sh.at[dst2_v.at[cidx]], sem, add=True)

    def wait_s(cidx, rows, sem):
        pltpu.make_async_copy(rows, agg_sh.at[dst2_v.at[cidx]], sem).wait()

    start_g(0, rows_a, sem_ga)

    def scatter(cidx, rows):
        pltpu.sync_copy(rows, agg_sh.at[dst_v.at[pl.ds(cidx * C, C)]],
                        add=True)

    def body(jj, carry):
        ca = 2 * jj
        cb = 2 * jj + 1
        start_g(cb, rows_b, sem_gb)
        wait_g(ca, rows_a, sem_ga)
        scatter(ca, rows_a)
        start_g(ca + 2, rows_a, sem_ga)
        wait_g(cb, rows_b, sem_gb)
        scatter(cb, rows_b)
        return carry

    lax.fori_loop(0, (NCHUNK - 1) // 2, body, 0)
    wait_g(NCHUNK - 1, rows_a, sem_ga)
    scatter(NCHUNK - 1, rows_a)
    plsc.subcore_barrier()

    for k in range(RPT // C):
        base = s * RPT + k * C
        pltpu.sync_copy(agg_sh.at[pl.ds(base, C)], rows_a)
        pltpu.sync_copy(rows_a, out_hbm.at[c, pl.ds(base, C)])


@functools.lru_cache(maxsize=None)
def _make_sc_call():
    return pl.kernel(
        _sc_body,
        mesh=plsc.VectorSubcoreMesh(core_axis_name="c", subcore_axis_name="s"),
        out_type=jax.ShapeDtypeStruct((NC, NP, D), jnp.float32),
        scratch_types=[
            pltpu.VMEM((EPW,), jnp.int32),
            pltpu.VMEM((SST,), jnp.int32),
            pltpu.VMEM((EPW,), jnp.int32),
            pltpu.VMEM((C, D), jnp.float32),
            pltpu.VMEM((C, D), jnp.float32),
            pltpu.VMEM_SHARED((NP, D), jnp.float32),
            pltpu.SemaphoreType.DMA,
            pltpu.SemaphoreType.DMA,
            pltpu.SemaphoreType.DMA,
            pltpu.SemaphoreType.DMA,
        ],
    )


def kernel(adj, feats, r, V0, a0, Wsl0, V1, a1, Wsl1):
    BN = 1000
    xw = pl.pallas_call(
        _xw_body,
        grid=(N // BN,),
        in_specs=[
            pl.BlockSpec((R, NB), lambda i: (0, 0)),
            pl.BlockSpec((NB, D, D), lambda i: (0, 0, 0)),
            pl.BlockSpec((BN, D), lambda i: (i, 0)),
        ],
        out_specs=pl.BlockSpec((R, BN, D), lambda i: (0, i, 0)),
        out_shape=jax.ShapeDtypeStruct((R, N, D), jnp.float32),
    )(a1, V1, feats)

    partials = _make_sc_call()(adj[0], adj[1], r, xw.reshape(N * R, D))

    out = pl.pallas_call(
        _final_body,
        grid=(N // BN,),
        in_specs=[
            pl.BlockSpec((BN, D), lambda i: (i, 0)),
            pl.BlockSpec((D, D), lambda i: (0, 0)),
            pl.BlockSpec((NC, BN, D), lambda i: (0, i, 0)),
        ],
        out_specs=pl.BlockSpec((BN, D), lambda i: (i, 0)),
        out_shape=jax.ShapeDtypeStruct((N, D), jnp.float32),
    )(feats, Wsl1, partials)
    return out


# speedup vs baseline: 1.3887x; 1.0510x over previous
import functools

import jax
import jax.numpy as jnp
from jax import lax
from jax.experimental import pallas as pl
from jax.experimental.pallas import tpu as pltpu
from jax.experimental.pallas import tpu_sc as plsc

N = 10000
E = 320000
D = 128
R = 16
NB = 8

NC = 2
NS = 16
NW = NC * NS
EPW = E // NW
C = 80
NCHUNK = EPW // C
NP = 10240
RPT = NP // NS
SST = 2000


def _xw_body(a_ref, v_ref, f_ref, out_ref):
    w = jax.lax.dot_general(a_ref[...], v_ref[...],
                            (((1,), (0,)), ((), ())),
                            preferred_element_type=jnp.float32)
    w = w.astype(jnp.bfloat16)
    f = f_ref[...].astype(jnp.bfloat16)
    for rr in range(R):
        out_ref[rr] = jnp.dot(f, w[rr], preferred_element_type=jnp.float32)


def _final_body(f_ref, w_ref, p_ref, out_ref):
    acc = p_ref[0] + p_ref[1] + jnp.dot(f_ref[...], w_ref[...],
                                        preferred_element_type=jnp.float32)
    out_ref[...] = jnp.maximum(acc, 0.0)


def _sc_body(adj_hbm, rel_hbm, xw_hbm, out_hbm,
             g_v, srcst_v, dst_v, rows_a, rows_b, agg_sh,
             sem_ga, sem_gb, sem_sa, sem_sb):
    c = lax.axis_index("c")
    s = lax.axis_index("s")
    wid = c * NS + s

    zero16 = jnp.zeros((16,), jnp.float32)

    def zrow(i, carry):
        for j in range(D // 16):
            rows_a[i, pl.ds(j * 16, 16)] = zero16
        return carry

    lax.fori_loop(0, C, zrow, 0)
    for k in range(RPT // C):
        pltpu.sync_copy(rows_a, agg_sh.at[pl.ds(s * RPT + k * C, C)])

    pltpu.sync_copy(rel_hbm.at[pl.ds(wid * EPW, EPW)], g_v)
    pltpu.sync_copy(adj_hbm.at[pl.ds(E + wid * EPW, EPW)], dst_v)
    for h in range(EPW // SST):
        pltpu.sync_copy(adj_hbm.at[pl.ds(wid * EPW + h * SST, SST)], srcst_v)

        def gstep(i, carry):
            sl = pl.ds(h * SST + i * 16, 16)
            g_v[sl] = g_v[sl] * N + srcst_v[pl.ds(i * 16, 16)]
            return carry

        lax.fori_loop(0, SST // 16, gstep, 0)
    plsc.subcore_barrier()

    def start_g(cidx, rows, sem):
        pltpu.async_copy(xw_hbm.at[g_v.at[pl.ds(cidx * C, C)]], rows, sem)

    def wait_g(cidx, rows, sem):
        pltpu.make_async_copy(xw_hbm.at[g_v.at[pl.ds(cidx * C, C)]], rows,
                              sem).wait()

    def start_s(cidx, rows, sem):
        pltpu.async_copy(rows, agg_sh.at[dst2_v.at[cidx]], sem, add=True)

    def wait_s(cidx, rows, sem):
        pltpu.make_async_copy(rows, agg_sh.at[dst2_v.at[cidx]], sem).wait()

    start_g(0, rows_a, sem_ga)

    def scatter(cidx, rows):
        pltpu.sync_copy(rows, agg_sh.at[dst_v.at[pl.ds(cidx * C, C)]],
                        add=True)

    def body(jj, carry):
        ca = 2 * jj
        cb = 2 * jj + 1
        start_g(cb, rows_b, sem_gb)
        wait_g(ca, rows_a, sem_ga)
        scatter(ca, rows_a)
        start_g(ca + 2, rows_a, sem_ga)
        wait_g(cb, rows_b, sem_gb)
        scatter(cb, rows_b)
        return carry

    lax.fori_loop(0, (NCHUNK - 1) // 2, body, 0)
    wait_g(NCHUNK - 1, rows_a, sem_ga)
    scatter(NCHUNK - 1, rows_a)
    plsc.subcore_barrier()

    for k in range(RPT // C):
        base = s * RPT + k * C
        pltpu.sync_copy(agg_sh.at[pl.ds(base, C)], rows_a)
        pltpu.sync_copy(rows_a, out_hbm.at[c, pl.ds(base, C)])


@functools.lru_cache(maxsize=None)
def _make_sc_call():
    return pl.kernel(
        _sc_body,
        mesh=plsc.VectorSubcoreMesh(core_axis_name="c", subcore_axis_name="s"),
        out_type=jax.ShapeDtypeStruct((NC, NP, D), jnp.float32),
        scratch_types=[
            pltpu.VMEM((EPW,), jnp.int32),
            pltpu.VMEM((SST,), jnp.int32),
            pltpu.VMEM((EPW,), jnp.int32),
            pltpu.VMEM((C, D), jnp.float32),
            pltpu.VMEM((C, D), jnp.float32),
            pltpu.VMEM_SHARED((NP, D), jnp.float32),
            pltpu.SemaphoreType.DMA,
            pltpu.SemaphoreType.DMA,
            pltpu.SemaphoreType.DMA,
            pltpu.SemaphoreType.DMA,
        ],
    )


def kernel(adj, feats, r, V0, a0, Wsl0, V1, a1, Wsl1):
    BN = 1000
    xw = pl.pallas_call(
        _xw_body,
        grid=(N // BN,),
        in_specs=[
            pl.BlockSpec((R, NB), lambda i: (0, 0)),
            pl.BlockSpec((NB, D, D), lambda i: (0, 0, 0)),
            pl.BlockSpec((BN, D), lambda i: (i, 0)),
        ],
        out_specs=pl.BlockSpec((R, BN, D), lambda i: (0, i, 0)),
        out_shape=jax.ShapeDtypeStruct((R, N, D), jnp.float32),
    )(a1, V1, feats)

    partials = _make_sc_call()(adj.reshape(2 * E), r, xw.reshape(N * R, D))

    out = pl.pallas_call(
        _final_body,
        grid=(N // BN,),
        in_specs=[
            pl.BlockSpec((BN, D), lambda i: (i, 0)),
            pl.BlockSpec((D, D), lambda i: (0, 0)),
            pl.BlockSpec((NC, BN, D), lambda i: (0, i, 0)),
        ],
        out_specs=pl.BlockSpec((BN, D), lambda i: (i, 0)),
        out_shape=jax.ShapeDtypeStruct((N, D), jnp.float32),
    )(feats, Wsl1, partials)
    return out


# speedup vs baseline: 1.5377x; 1.1073x over previous
import functools

import jax
import jax.numpy as jnp
from jax import lax
from jax.experimental import pallas as pl
from jax.experimental.pallas import tpu as pltpu
from jax.experimental.pallas import tpu_sc as plsc

N = 10000
E = 320000
D = 128
R = 16
NB = 8

NC = 2
NS = 16
NW = NC * NS
EPW = E // NW
C = 40
NCHUNK = EPW // C
NBUF = 4
NP = 10240
RPT = NP // NS
SST = 2000


def _xw_body(a_ref, v_ref, f_ref, out_ref):
    w = jax.lax.dot_general(a_ref[...], v_ref[...],
                            (((1,), (0,)), ((), ())),
                            preferred_element_type=jnp.float32)
    w = w.astype(jnp.bfloat16)
    f = f_ref[...].astype(jnp.bfloat16)
    for rr in range(R):
        out_ref[rr] = jnp.dot(f, w[rr], preferred_element_type=jnp.float32)


def _final_body(f_ref, w_ref, p_ref, out_ref):
    acc = p_ref[0] + p_ref[1] + jnp.dot(f_ref[...], w_ref[...],
                                        preferred_element_type=jnp.float32)
    out_ref[...] = jnp.maximum(acc, 0.0)


def _sc_body(adj_hbm, rel_hbm, xw_hbm, out_hbm,
             g_v, srcst_v, dst_v, rows_0, rows_1, rows_2, rows_3, agg_sh,
             sem_0, sem_1, sem_2, sem_3):
    rows = (rows_0, rows_1, rows_2, rows_3)
    sems = (sem_0, sem_1, sem_2, sem_3)
    c = lax.axis_index("c")
    s = lax.axis_index("s")
    wid = c * NS + s

    zero16 = jnp.zeros((16,), jnp.float32)

    def zrow(i, carry):
        for j in range(D // 16):
            rows_0[i, pl.ds(j * 16, 16)] = zero16
        return carry

    lax.fori_loop(0, C, zrow, 0)
    for k in range(RPT // C):
        pltpu.sync_copy(rows_0, agg_sh.at[pl.ds(s * RPT + k * C, C)])

    pltpu.sync_copy(rel_hbm.at[pl.ds(wid * EPW, EPW)], g_v)
    pltpu.sync_copy(adj_hbm.at[pl.ds(E + wid * EPW, EPW)], dst_v)
    for h in range(EPW // SST):
        pltpu.sync_copy(adj_hbm.at[pl.ds(wid * EPW + h * SST, SST)], srcst_v)

        def gstep(i, carry):
            sl = pl.ds(h * SST + i * 16, 16)
            g_v[sl] = g_v[sl] * N + srcst_v[pl.ds(i * 16, 16)]
            return carry

        lax.fori_loop(0, SST // 16, gstep, 0)
    plsc.subcore_barrier()

    def start_g(cidx, rbuf, sem):
        pltpu.async_copy(xw_hbm.at[g_v.at[pl.ds(cidx * C, C)]], rbuf, sem)

    def wait_g(cidx, rbuf, sem):
        pltpu.make_async_copy(xw_hbm.at[g_v.at[pl.ds(cidx * C, C)]], rbuf,
                              sem).wait()

    def scatter(cidx, rbuf):
        pltpu.sync_copy(rbuf, agg_sh.at[dst_v.at[pl.ds(cidx * C, C)]],
                        add=True)

    for k in range(NBUF):
        start_g(k, rows[k], sems[k])

    def body(jj, carry):
        for k in range(NBUF):
            cc = NBUF * jj + k
            wait_g(cc, rows[k], sems[k])
            scatter(cc, rows[k])

            @pl.when(cc + NBUF < NCHUNK)
            def _():
                start_g(cc + NBUF, rows[k], sems[k])
        return carry

    lax.fori_loop(0, NCHUNK // NBUF, body, 0)
    for cc in range((NCHUNK // NBUF) * NBUF, NCHUNK):
        wait_g(cc, rows[cc % NBUF], sems[cc % NBUF])
        scatter(cc, rows[cc % NBUF])
    plsc.subcore_barrier()

    for k in range(RPT // C):
        base = s * RPT + k * C
        rb = rows[k % NBUF]
        pltpu.sync_copy(agg_sh.at[pl.ds(base, C)], rb)
        pltpu.sync_copy(rb, out_hbm.at[c, pl.ds(base, C)])


@functools.lru_cache(maxsize=None)
def _make_sc_call():
    return pl.kernel(
        _sc_body,
        mesh=plsc.VectorSubcoreMesh(core_axis_name="c", subcore_axis_name="s"),
        out_type=jax.ShapeDtypeStruct((NC, NP, D), jnp.float32),
        scratch_types=[
            pltpu.VMEM((EPW,), jnp.int32),
            pltpu.VMEM((SST,), jnp.int32),
            pltpu.VMEM((EPW,), jnp.int32),
            pltpu.VMEM((C, D), jnp.float32),
            pltpu.VMEM((C, D), jnp.float32),
            pltpu.VMEM((C, D), jnp.float32),
            pltpu.VMEM((C, D), jnp.float32),
            pltpu.VMEM_SHARED((NP, D), jnp.float32),
            pltpu.SemaphoreType.DMA,
            pltpu.SemaphoreType.DMA,
            pltpu.SemaphoreType.DMA,
            pltpu.SemaphoreType.DMA,
        ],
    )


def kernel(adj, feats, r, V0, a0, Wsl0, V1, a1, Wsl1):
    BN = 1000
    xw = pl.pallas_call(
        _xw_body,
        grid=(N // BN,),
        in_specs=[
            pl.BlockSpec((R, NB), lambda i: (0, 0)),
            pl.BlockSpec((NB, D, D), lambda i: (0, 0, 0)),
            pl.BlockSpec((BN, D), lambda i: (i, 0)),
        ],
        out_specs=pl.BlockSpec((R, BN, D), lambda i: (0, i, 0)),
        out_shape=jax.ShapeDtypeStruct((R, N, D), jnp.float32),
    )(a1, V1, feats)

    partials = _make_sc_call()(adj.reshape(2 * E), r, xw.reshape(N * R, D))

    out = pl.pallas_call(
        _final_body,
        grid=(N // BN,),
        in_specs=[
            pl.BlockSpec((BN, D), lambda i: (i, 0)),
            pl.BlockSpec((D, D), lambda i: (0, 0)),
            pl.BlockSpec((NC, BN, D), lambda i: (0, i, 0)),
        ],
        out_specs=pl.BlockSpec((BN, D), lambda i: (i, 0)),
        out_shape=jax.ShapeDtypeStruct((N, D), jnp.float32),
    )(feats, Wsl1, partials)
    return out


# speedup vs baseline: 1.5874x; 1.0323x over previous
import functools

import jax
import jax.numpy as jnp
from jax import lax
from jax.experimental import pallas as pl
from jax.experimental.pallas import tpu as pltpu
from jax.experimental.pallas import tpu_sc as plsc

N = 10000
E = 320000
D = 128
R = 16
NB = 8

NC = 2
NS = 16
NW = NC * NS
EPW = E // NW
C = 40
NCHUNK = EPW // C
NBUF = 5
NP = 10240
RPT = NP // NS
SST = 2000


def _xw_body(a_ref, v_ref, f_ref, out_ref):
    w = jax.lax.dot_general(a_ref[...], v_ref[...],
                            (((1,), (0,)), ((), ())),
                            preferred_element_type=jnp.float32)
    w = w.astype(jnp.bfloat16)
    f = f_ref[...].astype(jnp.bfloat16)
    for rr in range(R):
        out_ref[rr] = jnp.dot(f, w[rr], preferred_element_type=jnp.float32)


def _final_body(f_ref, w_ref, p_ref, out_ref):
    acc = p_ref[0] + p_ref[1] + jnp.dot(f_ref[...], w_ref[...],
                                        preferred_element_type=jnp.float32)
    out_ref[...] = jnp.maximum(acc, 0.0)


def _sc_body(adj_hbm, rel_hbm, xw_hbm, out_hbm,
             g_v, srcst_v, dst_v, rows_0, rows_1, rows_2, rows_3, rows_4,
             agg_sh, sem_0, sem_1, sem_2, sem_3, sem_4):
    rows = (rows_0, rows_1, rows_2, rows_3, rows_4)
    sems = (sem_0, sem_1, sem_2, sem_3, sem_4)
    c = lax.axis_index("c")
    s = lax.axis_index("s")
    wid = c * NS + s

    zero16 = jnp.zeros((16,), jnp.float32)

    def zrow(i, carry):
        for j in range(D // 16):
            rows_0[i, pl.ds(j * 16, 16)] = zero16
        return carry

    lax.fori_loop(0, C, zrow, 0)
    for k in range(RPT // C):
        pltpu.sync_copy(rows_0, agg_sh.at[pl.ds(s * RPT + k * C, C)])

    pltpu.sync_copy(rel_hbm.at[pl.ds(wid * EPW, EPW)], g_v)
    pltpu.sync_copy(adj_hbm.at[pl.ds(E + wid * EPW, EPW)], dst_v)
    for h in range(EPW // SST):
        pltpu.sync_copy(adj_hbm.at[pl.ds(wid * EPW + h * SST, SST)], srcst_v)

        def gstep(i, carry):
            sl = pl.ds(h * SST + i * 16, 16)
            g_v[sl] = g_v[sl] * N + srcst_v[pl.ds(i * 16, 16)]
            return carry

        lax.fori_loop(0, SST // 16, gstep, 0)
    plsc.subcore_barrier()

    def start_g(cidx, rbuf, sem):
        pltpu.async_copy(xw_hbm.at[g_v.at[pl.ds(cidx * C, C)]], rbuf, sem)

    def wait_g(cidx, rbuf, sem):
        pltpu.make_async_copy(xw_hbm.at[g_v.at[pl.ds(cidx * C, C)]], rbuf,
                              sem).wait()

    def scatter(cidx, rbuf):
        pltpu.sync_copy(rbuf, agg_sh.at[dst_v.at[pl.ds(cidx * C, C)]],
                        add=True)

    for k in range(NBUF):
        start_g(k, rows[k], sems[k])

    def body(jj, carry):
        for k in range(NBUF):
            cc = NBUF * jj + k
            wait_g(cc, rows[k], sems[k])
            scatter(cc, rows[k])

            @pl.when(cc + NBUF < NCHUNK)
            def _():
                start_g(cc + NBUF, rows[k], sems[k])
        return carry

    lax.fori_loop(0, NCHUNK // NBUF, body, 0)
    for cc in range((NCHUNK // NBUF) * NBUF, NCHUNK):
        wait_g(cc, rows[cc % NBUF], sems[cc % NBUF])
        scatter(cc, rows[cc % NBUF])
    plsc.subcore_barrier()

    for k in range(RPT // C):
        base = s * RPT + k * C
        rb = rows[k % NBUF]
        pltpu.sync_copy(agg_sh.at[pl.ds(base, C)], rb)
        pltpu.sync_copy(rb, out_hbm.at[c, pl.ds(base, C)])


@functools.lru_cache(maxsize=None)
def _make_sc_call():
    return pl.kernel(
        _sc_body,
        mesh=plsc.VectorSubcoreMesh(core_axis_name="c", subcore_axis_name="s"),
        out_type=jax.ShapeDtypeStruct((NC, NP, D), jnp.float32),
        scratch_types=[
            pltpu.VMEM((EPW,), jnp.int32),
            pltpu.VMEM((SST,), jnp.int32),
            pltpu.VMEM((EPW,), jnp.int32),
            pltpu.VMEM((C, D), jnp.float32),
            pltpu.VMEM((C, D), jnp.float32),
            pltpu.VMEM((C, D), jnp.float32),
            pltpu.VMEM((C, D), jnp.float32),
            pltpu.VMEM((C, D), jnp.float32),
            pltpu.VMEM_SHARED((NP, D), jnp.float32),
            pltpu.SemaphoreType.DMA,
            pltpu.SemaphoreType.DMA,
            pltpu.SemaphoreType.DMA,
            pltpu.SemaphoreType.DMA,
            pltpu.SemaphoreType.DMA,
        ],
    )


def kernel(adj, feats, r, V0, a0, Wsl0, V1, a1, Wsl1):
    BN = 1000
    xw = pl.pallas_call(
        _xw_body,
        grid=(N // BN,),
        in_specs=[
            pl.BlockSpec((R, NB), lambda i: (0, 0)),
            pl.BlockSpec((NB, D, D), lambda i: (0, 0, 0)),
            pl.BlockSpec((BN, D), lambda i: (i, 0)),
        ],
        out_specs=pl.BlockSpec((R, BN, D), lambda i: (0, i, 0)),
        out_shape=jax.ShapeDtypeStruct((R, N, D), jnp.float32),
    )(a1, V1, feats)

    partials = _make_sc_call()(adj.reshape(2 * E), r, xw.reshape(N * R, D))

    out = pl.pallas_call(
        _final_body,
        grid=(N // BN,),
        in_specs=[
            pl.BlockSpec((BN, D), lambda i: (i, 0)),
            pl.BlockSpec((D, D), lambda i: (0, 0)),
            pl.BlockSpec((NC, BN, D), lambda i: (0, i, 0)),
        ],
        out_specs=pl.BlockSpec((BN, D), lambda i: (i, 0)),
        out_shape=jax.ShapeDtypeStruct((N, D), jnp.float32),
    )(feats, Wsl1, partials)
    return out


# speedup vs baseline: 1.6113x; 1.0150x over previous
import functools

import jax
import jax.numpy as jnp
from jax import lax
from jax.experimental import pallas as pl
from jax.experimental.pallas import tpu as pltpu
from jax.experimental.pallas import tpu_sc as plsc

N = 10000
E = 320000
D = 128
R = 16
NB = 8

NC = 2
NS = 16
NW = NC * NS
EPW = E // NW
C = 40
NCHUNK = EPW // C
NBUF = 5
NP = 10240
RPT = NP // NS
SST = 2000


def _xw_body(a_ref, v_ref, f_ref, out_ref):
    w = jax.lax.dot_general(a_ref[...], v_ref[...],
                            (((1,), (0,)), ((), ())),
                            preferred_element_type=jnp.float32)
    w = w.astype(jnp.bfloat16)
    f = f_ref[...].astype(jnp.bfloat16)
    for rr in range(R):
        out_ref[rr] = jnp.dot(f, w[rr], preferred_element_type=jnp.float32)


def _final_body(f_ref, w_ref, p_ref, out_ref):
    acc = p_ref[0] + p_ref[1] + jnp.dot(f_ref[...], w_ref[...],
                                        preferred_element_type=jnp.float32)
    out_ref[...] = jnp.maximum(acc, 0.0)


def _sc_body(adj_hbm, rel_hbm, xw_hbm, out_hbm,
             g_v, srcst_v, dst_v, rows_0, rows_1, rows_2, rows_3, rows_4,
             agg_sh, sem_0, sem_1, sem_2, sem_3, sem_4):
    rows = (rows_0, rows_1, rows_2, rows_3, rows_4)
    sems = (sem_0, sem_1, sem_2, sem_3, sem_4)
    c = lax.axis_index("c")
    s = lax.axis_index("s")
    wid = c * NS + s

    zero16 = jnp.zeros((16,), jnp.float32)

    def zrow(i, carry):
        for j in range(D // 16):
            rows_0[i, pl.ds(j * 16, 16)] = zero16
        return carry

    lax.fori_loop(0, C, zrow, 0)
    for k in range(RPT // C):
        pltpu.sync_copy(rows_0, agg_sh.at[pl.ds(s * RPT + k * C, C)])

    pltpu.sync_copy(rel_hbm.at[pl.ds(wid * EPW, EPW)], g_v)
    pltpu.sync_copy(adj_hbm.at[pl.ds(E + wid * EPW, EPW)], dst_v)
    for h in range(EPW // SST):
        pltpu.sync_copy(adj_hbm.at[pl.ds(wid * EPW + h * SST, SST)], srcst_v)

        def gstep(i, carry):
            sl = pl.ds(h * SST + i * 16, 16)
            g_v[sl] = g_v[sl] * N + srcst_v[pl.ds(i * 16, 16)]
            return carry

        lax.fori_loop(0, SST // 16, gstep, 0)
    plsc.subcore_barrier()

    def start_g(cidx, rbuf, sem):
        pltpu.async_copy(xw_hbm.at[g_v.at[pl.ds(cidx * C, C)]], rbuf, sem)

    def wait_g(cidx, rbuf, sem):
        pltpu.make_async_copy(xw_hbm.at[g_v.at[pl.ds(cidx * C, C)]], rbuf,
                              sem).wait()

    def scatter(cidx, rbuf):
        pltpu.sync_copy(rbuf, agg_sh.at[dst_v.at[pl.ds(cidx * C, C)]],
                        add=True)

    for k in range(NBUF):
        start_g(k, rows[k], sems[k])

    def body(jj, carry):
        for k in range(NBUF):
            cc = NBUF * jj + k
            wait_g(cc, rows[k], sems[k])
            scatter(cc, rows[k])

            @pl.when(cc + NBUF < NCHUNK)
            def _():
                start_g(cc + NBUF, rows[k], sems[k])
        return carry

    lax.fori_loop(0, NCHUNK // NBUF, body, 0)
    for cc in range((NCHUNK // NBUF) * NBUF, NCHUNK):
        wait_g(cc, rows[cc % NBUF], sems[cc % NBUF])
        scatter(cc, rows[cc % NBUF])
    plsc.subcore_barrier()

    pltpu.sync_copy(agg_sh.at[pl.ds(s * RPT, RPT)],
                    out_hbm.at[c, pl.ds(s * RPT, RPT)])


@functools.lru_cache(maxsize=None)
def _make_sc_call():
    return pl.kernel(
        _sc_body,
        mesh=plsc.VectorSubcoreMesh(core_axis_name="c", subcore_axis_name="s"),
        out_type=jax.ShapeDtypeStruct((NC, NP, D), jnp.float32),
        scratch_types=[
            pltpu.VMEM((EPW,), jnp.int32),
            pltpu.VMEM((SST,), jnp.int32),
            pltpu.VMEM((EPW,), jnp.int32),
            pltpu.VMEM((C, D), jnp.float32),
            pltpu.VMEM((C, D), jnp.float32),
            pltpu.VMEM((C, D), jnp.float32),
            pltpu.VMEM((C, D), jnp.float32),
            pltpu.VMEM((C, D), jnp.float32),
            pltpu.VMEM_SHARED((NP, D), jnp.float32),
            pltpu.SemaphoreType.DMA,
            pltpu.SemaphoreType.DMA,
            pltpu.SemaphoreType.DMA,
            pltpu.SemaphoreType.DMA,
            pltpu.SemaphoreType.DMA,
        ],
    )


def kernel(adj, feats, r, V0, a0, Wsl0, V1, a1, Wsl1):
    BN = 1000
    xw = pl.pallas_call(
        _xw_body,
        grid=(N // BN,),
        in_specs=[
            pl.BlockSpec((R, NB), lambda i: (0, 0)),
            pl.BlockSpec((NB, D, D), lambda i: (0, 0, 0)),
            pl.BlockSpec((BN, D), lambda i: (i, 0)),
        ],
        out_specs=pl.BlockSpec((R, BN, D), lambda i: (0, i, 0)),
        out_shape=jax.ShapeDtypeStruct((R, N, D), jnp.float32),
    )(a1, V1, feats)

    partials = _make_sc_call()(adj.reshape(2 * E), r, xw.reshape(N * R, D))

    out = pl.pallas_call(
        _final_body,
        grid=(N // BN,),
        in_specs=[
            pl.BlockSpec((BN, D), lambda i: (i, 0)),
            pl.BlockSpec((D, D), lambda i: (0, 0)),
            pl.BlockSpec((NC, BN, D), lambda i: (0, i, 0)),
        ],
        out_specs=pl.BlockSpec((BN, D), lambda i: (i, 0)),
        out_shape=jax.ShapeDtypeStruct((N, D), jnp.float32),
    )(feats, Wsl1, partials)
    return out
